# baseline (speedup 1.0000x reference)
import functools

import jax
import jax.numpy as jnp
import numpy as np
from jax import lax
from jax.experimental import pallas as pl
from jax.experimental.pallas import tpu as pltpu
from jax.experimental.pallas import tpu_sc as plsc

N = 10000
E = 320000
V = 30522
S = 8
D = 768
H = 128
NG = 128
NCLS = 8

NC = 2
NS = 16
NW = NC * NS
NPAD = 10240
RPT = NPAD // NS
EPW = E // NW
CED = 200
CEM = 100
CET = 128
TOK = NPAD * S

_NREP2 = (np.arange(TOK, dtype=np.int32) % NPAD).reshape(
    TOK // CET // NW, NW, CET).swapaxes(0, 1).reshape(TOK // CET, CET)


def _proj_body(t_ref, w_ref, o_ref):
    o_ref[...] = jnp.dot(t_ref[...], w_ref[...],
                         preferred_element_type=jnp.float32)


def _proj(tbl, w1):
    bm = 1536
    return pl.pallas_call(
        _proj_body,
        grid=(pl.cdiv(V, bm),),
        in_specs=[pl.BlockSpec((bm, D), lambda i: (i, 0)),
                  pl.BlockSpec((D, H), lambda i: (0, 0))],
        out_specs=pl.BlockSpec((bm, H), lambda i: (i, 0)),
        out_shape=jax.ShapeDtypeStruct((V, H), jnp.float32),
    )(tbl, w1)


def _prep_body(deg_ref, t_ref, dv_ref, o_ref):
    d = deg_ref[0] + deg_ref[1] + 1.0
    dv = lax.rsqrt(jnp.maximum(d, 1.0))
    dv_ref[...] = dv
    o_ref[...] = (t_ref[0] + t_ref[1]) * (dv[:, :1] * (1.0 / S))


def _prep(deg, tokacc):
    bm = 2048
    return pl.pallas_call(
        _prep_body,
        grid=(NPAD // bm,),
        in_specs=[pl.BlockSpec((NC, bm, 16), lambda i: (0, i, 0)),
                  pl.BlockSpec((NC, bm, H), lambda i: (0, i, 0))],
        out_specs=[pl.BlockSpec((bm, 16), lambda i: (i, 0)),
                   pl.BlockSpec((bm, H), lambda i: (i, 0))],
        out_shape=(jax.ShapeDtypeStruct((NPAD, 16), jnp.float32),
                   jax.ShapeDtypeStruct((NPAD, H), jnp.float32)),
    )(deg, tokacc)


def _mm2_body(acc_ref, hws_ref, dv_ref, w2_ref, b1_ref, o_ref):
    dv = dv_ref[:, :1]
    a = acc_ref[0] + acc_ref[1] + hws_ref[...]
    h1 = jnp.maximum(a * dv + b1_ref[...], 0.0)
    o_ref[...] = jnp.dot(h1, w2_ref[...],
                         preferred_element_type=jnp.float32) * dv


def _mm2(parts, hws, dinv16, w2, b1r):
    bm = 2048
    return pl.pallas_call(
        _mm2_body,
        grid=(NPAD // bm,),
        in_specs=[pl.BlockSpec((NC, bm, H), lambda i: (0, i, 0)),
                  pl.BlockSpec((bm, H), lambda i: (i, 0)),
                  pl.BlockSpec((bm, 16), lambda i: (i, 0)),
                  pl.BlockSpec((H, H), lambda i: (0, 0)),
                  pl.BlockSpec((1, H), lambda i: (0, 0))],
        out_specs=pl.BlockSpec((bm, H), lambda i: (i, 0)),
        out_shape=jax.ShapeDtypeStruct((NPAD, H), jnp.float32),
    )(parts, hws, dinv16, w2, b1r)


def _pool_body(acc_ref, hws_ref, dv_ref, b_ref, b2_ref, wfc_ref, bfc_ref,
               o_ref, g_ref, cnt_ref):
    k = pl.program_id(0)

    @pl.when(k == 0)
    def _():
        g_ref[...] = jnp.zeros_like(g_ref)
        cnt_ref[...] = jnp.zeros_like(cnt_ref)

    ids = b_ref[...]
    eq = (ids[None, :] == lax.broadcasted_iota(jnp.int32, (NG, ids.shape[0]),
                                               0)).astype(jnp.float32)
    h = (acc_ref[0] + acc_ref[1] + hws_ref[...]) * dv_ref[:, :1]
    g_ref[...] += jnp.dot(eq, h, preferred_element_type=jnp.float32)
    cnt_ref[...] += jnp.sum(eq, axis=1, keepdims=True)

    @pl.when(k == pl.num_programs(0) - 1)
    def _():
        cnt = cnt_ref[...]
        g = (g_ref[...] / jnp.maximum(cnt, 1.0)
             + b2_ref[...] * (cnt > 0.0).astype(jnp.float32))
        o_ref[...] = jnp.dot(g, wfc_ref[...],
                             preferred_element_type=jnp.float32) + bfc_ref[...]


def _pool(parts, hws, dinv16, batp, b2r, wfc, bfcr):
    bk = 2048
    return pl.pallas_call(
        _pool_body,
        grid=(NPAD // bk,),
        in_specs=[pl.BlockSpec((NC, bk, H), lambda i: (0, i, 0)),
                  pl.BlockSpec((bk, H), lambda i: (i, 0)),
                  pl.BlockSpec((bk, 16), lambda i: (i, 0)),
                  pl.BlockSpec((bk,), lambda i: (i,)),
                  pl.BlockSpec((1, H), lambda i: (0, 0)),
                  pl.BlockSpec((H, NCLS), lambda i: (0, 0)),
                  pl.BlockSpec((1, NCLS), lambda i: (0, 0))],
        out_specs=pl.BlockSpec((NG, NCLS), lambda i: (0, 0)),
        out_shape=jax.ShapeDtypeStruct((NG, NCLS), jnp.float32),
        scratch_shapes=[pltpu.VMEM((NG, H), jnp.float32),
                        pltpu.VMEM((NG, 1), jnp.float32)],
    )(parts, hws, dinv16, batp, b2r, wfc, bfcr)


@functools.cache
def _sc_kernels():
    mesh = plsc.VectorSubcoreMesh(core_axis_name="c", subcore_axis_name="s",
                                  num_cores=NC, num_subcores=NS)

    ndch = EPW // CED

    @functools.partial(
        pl.kernel,
        compiler_params=pltpu.CompilerParams(use_tc_tiling_on_sc=False),
        out_type=jax.ShapeDtypeStruct((NC, NPAD, 16), jnp.float32),
        mesh=mesh,
        scratch_types=[
            pltpu.VMEM((ndch, CED), jnp.int32),
            pltpu.VMEM((CED, 16), jnp.float32),
            pltpu.VMEM((RPT, 16), jnp.float32),
            pltpu.VMEM_SHARED((NPAD, 16), jnp.float32),
            pltpu.SemaphoreType.DMA,
        ],
    )
    def _deg(dst2_hbm, out_hbm, di_v, ones_v, zb_v, acc_sh, semd):
        c = lax.axis_index("c")
        s = lax.axis_index("s")
        wid = s * NC + c

        pltpu.sync_copy(dst2_hbm.at[pl.ds(wid * ndch, ndch)], di_v)

        @pl.loop(0, CED)
        def _fill(r):
            ones_v[r] = jnp.ones((16,), jnp.float32)

        @pl.loop(0, RPT)
        def _zero(r):
            zb_v[r] = jnp.zeros((16,), jnp.float32)

        pltpu.sync_copy(zb_v, acc_sh.at[pl.ds(s * RPT, RPT)])
        plsc.subcore_barrier()

        @pl.loop(0, ndch)
        def _chunk(i):
            pltpu.async_copy(ones_v, acc_sh.at[di_v.at[i]], semd, add=True)

        @pl.loop(0, ndch)
        def _drain(i):
            pltpu.make_async_copy(ones_v, acc_sh.at[di_v.at[i]], semd).wait()

        plsc.subcore_barrier()
        pltpu.sync_copy(acc_sh.at[pl.ds(s * RPT, RPT)],
                        out_hbm.at[c, pl.ds(s * RPT, RPT)])

    def _make_scat(tot, ce, name):
        nch = tot // NW // ce
        assert tot % (NW * ce) == 0 and nch % 2 == 0
        zsegs = []
        off = 0
        while off < RPT:
            step = min(ce, RPT - off)
            zsegs.append((off, step))
            off += step

        @functools.partial(
            pl.kernel,
            compiler_params=pltpu.CompilerParams(use_tc_tiling_on_sc=False),
            out_type=jax.ShapeDtypeStruct((NC, NPAD, H), jnp.float32),
            mesh=mesh,
            scratch_types=[
                pltpu.VMEM((nch, ce), jnp.int32),
                pltpu.VMEM((nch, ce), jnp.int32),
                pltpu.VMEM((ce, H), jnp.float32),
                pltpu.VMEM((ce, H), jnp.float32),
                pltpu.VMEM_SHARED((NPAD, H), jnp.float32),
                pltpu.SemaphoreType.DMA,
                pltpu.SemaphoreType.DMA,
            ],
            name=name,
        )
        def _scat(tbl_hbm, si2_hbm, di2_hbm, out_hbm, si_v, di_v,
                  rows0_v, rows1_v, acc_sh, sem0, sem1):
            c = lax.axis_index("c")
            s = lax.axis_index("s")
            wid = s * NC + c

            pltpu.sync_copy(si2_hbm.at[pl.ds(wid * nch, nch)], si_v)
            pltpu.sync_copy(di2_hbm.at[pl.ds(wid * nch, nch)], di_v)

            @pl.loop(0, ce)
            def _z(r):
                for k in range(H // 16):
                    rows0_v[r, pl.ds(16 * k, 16)] = jnp.zeros((16,),
                                                              jnp.float32)

            for off, step in zsegs:
                pltpu.sync_copy(rows0_v.at[pl.ds(0, step)],
                                acc_sh.at[pl.ds(s * RPT + off, step)])
            plsc.subcore_barrier()

            pltpu.async_copy(tbl_hbm.at[si_v.at[0]], rows0_v, sem0)
            pltpu.async_copy(tbl_hbm.at[si_v.at[1]], rows1_v, sem1)

            @pl.loop(0, nch // 2)
            def _chunk(i2):
                i = i2 * 2
                pltpu.make_async_copy(tbl_hbm.at[si_v.at[i]], rows0_v,
                                      sem0).wait()
                pltpu.sync_copy(rows0_v, acc_sh.at[di_v.at[i]], add=True)

                @pl.when(i + 2 < nch)
                def _():
                    pltpu.async_copy(tbl_hbm.at[si_v.at[i + 2]], rows0_v, sem0)

                pltpu.make_async_copy(tbl_hbm.at[si_v.at[i + 1]], rows1_v,
                                      sem1).wait()
                pltpu.sync_copy(rows1_v, acc_sh.at[di_v.at[i + 1]], add=True)

                @pl.when(i + 3 < nch)
                def _():
                    pltpu.async_copy(tbl_hbm.at[si_v.at[i + 3]], rows1_v, sem1)

            plsc.subcore_barrier()
            pltpu.sync_copy(acc_sh.at[pl.ds(s * RPT, RPT)],
                            out_hbm.at[c, pl.ds(s * RPT, RPT)])

        return _scat

    return (_deg, _make_scat(TOK, CET, "sc_tok_scatter"),
            _make_scat(E, CEM, "sc_msg_scatter"))


def kernel(x, edge_index, batch, emb_table, W1, b1, W2, b2, Wfc, bfc):
    x = x.astype(jnp.int32)
    ei = edge_index.astype(jnp.int32)
    bat = batch.astype(jnp.int32)

    ntch = TOK // CET
    xf2 = jnp.pad(x, ((0, NPAD - N), (0, 0))).T.reshape(ntch // NW, NW, CET)
    xf2 = xf2.swapaxes(0, 1).reshape(ntch, CET)
    nrep2 = jnp.asarray(_NREP2)
    src2 = ei[0].reshape(E // CEM, CEM)
    dst2 = ei[1].reshape(E // CEM, CEM)
    batp = jnp.pad(bat, (0, NPAD - N), constant_values=-1)

    _deg, _scat_tok, _scat_msg = _sc_kernels()
    p = _proj(emb_table, W1)
    deg = _deg(ei[1].reshape(E // CED, CED))
    tokacc = _scat_tok(p, xf2, nrep2)
    dinv16, hw1s = _prep(deg, tokacc)
    parts1 = _scat_msg(hw1s, src2, dst2)
    hw2s = _mm2(parts1, hw1s, dinv16, W2, b1.reshape(1, H))
    parts2 = _scat_msg(hw2s, src2, dst2)
    return _pool(parts2, hw2s, dinv16, batp, b2.reshape(1, H), Wfc,
                 bfc.reshape(1, NCLS))


# speedup vs baseline: 1.0060x; 1.0060x over previous
import functools

import jax
import jax.numpy as jnp
from jax import lax
from jax.experimental import pallas as pl
from jax.experimental.pallas import tpu as pltpu
from jax.experimental.pallas import tpu_sc as plsc

N = 10000
E = 320000
V = 30522
S = 8
D = 768
H = 128
NG = 128
NCLS = 8

NC = 2
NS = 16
NW = NC * NS
NPAD = 10240
RPT = NPAD // NS
EPW = E // NW
CED = 200
CEM = 100
CET = 128
TOK = NPAD * S


def _proj_body(t_ref, w_ref, o_ref):
    o_ref[...] = jnp.dot(t_ref[...], w_ref[...],
                         preferred_element_type=jnp.float32)


def _proj(tbl, w1):
    bm = 1536
    return pl.pallas_call(
        _proj_body,
        grid=(pl.cdiv(V, bm),),
        in_specs=[pl.BlockSpec((bm, D), lambda i: (i, 0)),
                  pl.BlockSpec((D, H), lambda i: (0, 0))],
        out_specs=pl.BlockSpec((bm, H), lambda i: (i, 0)),
        out_shape=jax.ShapeDtypeStruct((V, H), jnp.float32),
    )(tbl, w1)


def _prep_body(deg_ref, t_ref, dv_ref, o_ref):
    d = deg_ref[0] + deg_ref[1] + 1.0
    dv = lax.rsqrt(jnp.maximum(d, 1.0))
    dv_ref[...] = dv
    o_ref[...] = (t_ref[0] + t_ref[1]) * (dv[:, :1] * (1.0 / S))


def _prep(deg, tokacc):
    bm = 2048
    return pl.pallas_call(
        _prep_body,
        grid=(NPAD // bm,),
        in_specs=[pl.BlockSpec((NC, bm, 16), lambda i: (0, i, 0)),
                  pl.BlockSpec((NC, bm, H), lambda i: (0, i, 0))],
        out_specs=[pl.BlockSpec((bm, 16), lambda i: (i, 0)),
                   pl.BlockSpec((bm, H), lambda i: (i, 0))],
        out_shape=(jax.ShapeDtypeStruct((NPAD, 16), jnp.float32),
                   jax.ShapeDtypeStruct((NPAD, H), jnp.float32)),
    )(deg, tokacc)


def _mm2_body(acc_ref, hws_ref, dv_ref, w2_ref, b1_ref, o_ref):
    dv = dv_ref[:, :1]
    a = acc_ref[0] + acc_ref[1] + hws_ref[...]
    h1 = jnp.maximum(a * dv + b1_ref[...], 0.0)
    o_ref[...] = jnp.dot(h1, w2_ref[...],
                         preferred_element_type=jnp.float32) * dv


def _mm2(parts, hws, dinv16, w2, b1r):
    bm = 2048
    return pl.pallas_call(
        _mm2_body,
        grid=(NPAD // bm,),
        in_specs=[pl.BlockSpec((NC, bm, H), lambda i: (0, i, 0)),
                  pl.BlockSpec((bm, H), lambda i: (i, 0)),
                  pl.BlockSpec((bm, 16), lambda i: (i, 0)),
                  pl.BlockSpec((H, H), lambda i: (0, 0)),
                  pl.BlockSpec((1, H), lambda i: (0, 0))],
        out_specs=pl.BlockSpec((bm, H), lambda i: (i, 0)),
        out_shape=jax.ShapeDtypeStruct((NPAD, H), jnp.float32),
    )(parts, hws, dinv16, w2, b1r)


def _pool_body(acc_ref, hws_ref, dv_ref, b_ref, b2_ref, wfc_ref, bfc_ref,
               o_ref, g_ref, cnt_ref):
    k = pl.program_id(0)

    @pl.when(k == 0)
    def _():
        g_ref[...] = jnp.zeros_like(g_ref)
        cnt_ref[...] = jnp.zeros_like(cnt_ref)

    ids = b_ref[...]
    eq = (ids[None, :] == lax.broadcasted_iota(jnp.int32, (NG, ids.shape[0]),
                                               0)).astype(jnp.float32)
    h = (acc_ref[0] + acc_ref[1] + hws_ref[...]) * dv_ref[:, :1]
    g_ref[...] += jnp.dot(eq, h, preferred_element_type=jnp.float32)
    cnt_ref[...] += jnp.sum(eq, axis=1, keepdims=True)

    @pl.when(k == pl.num_programs(0) - 1)
    def _():
        cnt = cnt_ref[...]
        g = (g_ref[...] / jnp.maximum(cnt, 1.0)
             + b2_ref[...] * (cnt > 0.0).astype(jnp.float32))
        o_ref[...] = jnp.dot(g, wfc_ref[...],
                             preferred_element_type=jnp.float32) + bfc_ref[...]


def _pool(parts, hws, dinv16, batp, b2r, wfc, bfcr):
    bk = 2048
    return pl.pallas_call(
        _pool_body,
        grid=(NPAD // bk,),
        in_specs=[pl.BlockSpec((NC, bk, H), lambda i: (0, i, 0)),
                  pl.BlockSpec((bk, H), lambda i: (i, 0)),
                  pl.BlockSpec((bk, 16), lambda i: (i, 0)),
                  pl.BlockSpec((bk,), lambda i: (i,)),
                  pl.BlockSpec((1, H), lambda i: (0, 0)),
                  pl.BlockSpec((H, NCLS), lambda i: (0, 0)),
                  pl.BlockSpec((1, NCLS), lambda i: (0, 0))],
        out_specs=pl.BlockSpec((NG, NCLS), lambda i: (0, 0)),
        out_shape=jax.ShapeDtypeStruct((NG, NCLS), jnp.float32),
        scratch_shapes=[pltpu.VMEM((NG, H), jnp.float32),
                        pltpu.VMEM((NG, 1), jnp.float32)],
    )(parts, hws, dinv16, batp, b2r, wfc, bfcr)


@functools.cache
def _sc_kernels():
    mesh = plsc.VectorSubcoreMesh(core_axis_name="c", subcore_axis_name="s",
                                  num_cores=NC, num_subcores=NS)

    ndch = EPW // CED

    @functools.partial(
        pl.kernel,
        compiler_params=pltpu.CompilerParams(use_tc_tiling_on_sc=False),
        out_type=jax.ShapeDtypeStruct((NC, NPAD, 16), jnp.float32),
        mesh=mesh,
        scratch_types=[
            pltpu.VMEM((ndch, CED), jnp.int32),
            pltpu.VMEM((CED, 16), jnp.float32),
            pltpu.VMEM((RPT, 16), jnp.float32),
            pltpu.VMEM_SHARED((NPAD, 16), jnp.float32),
            pltpu.SemaphoreType.DMA,
        ],
    )
    def _deg(dst2_hbm, out_hbm, di_v, ones_v, zb_v, acc_sh, semd):
        c = lax.axis_index("c")
        s = lax.axis_index("s")
        wid = s * NC + c

        pltpu.sync_copy(dst2_hbm.at[pl.ds(wid * ndch, ndch)], di_v)

        @pl.loop(0, CED)
        def _fill(r):
            ones_v[r] = jnp.ones((16,), jnp.float32)

        @pl.loop(0, RPT)
        def _zero(r):
            zb_v[r] = jnp.zeros((16,), jnp.float32)

        pltpu.sync_copy(zb_v, acc_sh.at[pl.ds(s * RPT, RPT)])
        plsc.subcore_barrier()

        @pl.loop(0, ndch)
        def _chunk(i):
            pltpu.async_copy(ones_v, acc_sh.at[di_v.at[i]], semd, add=True)

        @pl.loop(0, ndch)
        def _drain(i):
            pltpu.make_async_copy(ones_v, acc_sh.at[di_v.at[i]], semd).wait()

        plsc.subcore_barrier()
        pltpu.sync_copy(acc_sh.at[pl.ds(s * RPT, RPT)],
                        out_hbm.at[c, pl.ds(s * RPT, RPT)])

    def _make_scat(tot, ce, name):
        nch = tot // NW // ce
        assert tot % (NW * ce) == 0 and nch % 2 == 0
        zsegs = []
        off = 0
        while off < RPT:
            step = min(ce, RPT - off)
            zsegs.append((off, step))
            off += step

        @functools.partial(
            pl.kernel,
            compiler_params=pltpu.CompilerParams(use_tc_tiling_on_sc=False),
            out_type=jax.ShapeDtypeStruct((NC, NPAD, H), jnp.float32),
            mesh=mesh,
            scratch_types=[
                pltpu.VMEM((nch, ce), jnp.int32),
                pltpu.VMEM((nch, ce), jnp.int32),
                pltpu.VMEM((ce, H), jnp.float32),
                pltpu.VMEM((ce, H), jnp.float32),
                pltpu.VMEM_SHARED((NPAD, H), jnp.float32),
                pltpu.SemaphoreType.DMA,
                pltpu.SemaphoreType.DMA,
            ],
            name=name,
        )
        def _scat(tbl_hbm, si2_hbm, di2_hbm, out_hbm, si_v, di_v,
                  rows0_v, rows1_v, acc_sh, sem0, sem1):
            c = lax.axis_index("c")
            s = lax.axis_index("s")
            wid = s * NC + c

            pltpu.sync_copy(si2_hbm.at[pl.ds(wid * nch, nch)], si_v)
            pltpu.sync_copy(di2_hbm.at[pl.ds(wid * nch, nch)], di_v)

            @pl.loop(0, ce)
            def _z(r):
                for k in range(H // 16):
                    rows0_v[r, pl.ds(16 * k, 16)] = jnp.zeros((16,),
                                                              jnp.float32)

            for off, step in zsegs:
                pltpu.sync_copy(rows0_v.at[pl.ds(0, step)],
                                acc_sh.at[pl.ds(s * RPT + off, step)])
            plsc.subcore_barrier()

            pltpu.async_copy(tbl_hbm.at[si_v.at[0]], rows0_v, sem0)
            pltpu.async_copy(tbl_hbm.at[si_v.at[1]], rows1_v, sem1)

            @pl.loop(0, nch // 2)
            def _chunk(i2):
                i = i2 * 2
                pltpu.make_async_copy(tbl_hbm.at[si_v.at[i]], rows0_v,
                                      sem0).wait()
                pltpu.sync_copy(rows0_v, acc_sh.at[di_v.at[i]], add=True)

                @pl.when(i + 2 < nch)
                def _():
                    pltpu.async_copy(tbl_hbm.at[si_v.at[i + 2]], rows0_v, sem0)

                pltpu.make_async_copy(tbl_hbm.at[si_v.at[i + 1]], rows1_v,
                                      sem1).wait()
                pltpu.sync_copy(rows1_v, acc_sh.at[di_v.at[i + 1]], add=True)

                @pl.when(i + 3 < nch)
                def _():
                    pltpu.async_copy(tbl_hbm.at[si_v.at[i + 3]], rows1_v, sem1)

            plsc.subcore_barrier()
            pltpu.sync_copy(acc_sh.at[pl.ds(s * RPT, RPT)],
                            out_hbm.at[c, pl.ds(s * RPT, RPT)])

        return _scat

    return (_deg, _make_scat(TOK, CET, "sc_tok_scatter"),
            _make_scat(E, CEM, "sc_msg_scatter"))


def kernel(x, edge_index, batch, emb_table, W1, b1, W2, b2, Wfc, bfc):
    x = x.astype(jnp.int32)
    ei = edge_index.astype(jnp.int32)
    bat = batch.astype(jnp.int32)

    ntch = TOK // CET
    xf2 = jnp.pad(x, ((0, NPAD - N), (0, 0))).T.reshape(ntch // NW, NW, CET)
    xf2 = xf2.swapaxes(0, 1).reshape(ntch, CET)
    nrep2 = (jnp.arange(TOK, dtype=jnp.int32) % NPAD).reshape(
        ntch // NW, NW, CET).swapaxes(0, 1).reshape(ntch, CET)
    src2 = ei[0].reshape(E // CEM, CEM)
    dst2 = ei[1].reshape(E // CEM, CEM)
    batp = jnp.pad(bat, (0, NPAD - N), constant_values=-1)

    _deg, _scat_tok, _scat_msg = _sc_kernels()
    p = _proj(emb_table, W1)
    deg = _deg(ei[1].reshape(E // CED, CED))
    tokacc = _scat_tok(p, xf2, nrep2)
    dinv16, hw1s = _prep(deg, tokacc)
    parts1 = _scat_msg(hw1s, src2, dst2)
    hw2s = _mm2(parts1, hw1s, dinv16, W2, b1.reshape(1, H))
    parts2 = _scat_msg(hw2s, src2, dst2)
    return _pool(parts2, hw2s, dinv16, batp, b2.reshape(1, H), Wfc,
                 bfc.reshape(1, NCLS))
